# baseline (speedup 1.0000x reference)
import jax
import jax.numpy as jnp
from jax.experimental import pallas as pl
from jax.experimental.pallas import tpu as pltpu
from jax.experimental.pallas import tpu_sc as plsc

_LANES = 16
_BLOCK = 16384


def kernel(x, value):
    orig_shape = x.shape
    x1 = x.reshape(-1)
    n = x1.shape[0]
    v16 = jnp.broadcast_to(value, (_LANES,))
    mesh = plsc.VectorSubcoreMesh(core_axis_name="c", subcore_axis_name="s")

    @pl.kernel(
        out_type=jax.ShapeDtypeStruct((n,), x.dtype),
        mesh=mesh,
        scratch_types=[pltpu.VMEM((_LANES,), jnp.float32)],
    )
    def _sc_scale(x_hbm, v_hbm, o_hbm, v_vmem):
        pltpu.sync_copy(v_hbm, v_vmem)

        def body(x_vmem, o_vmem):
            inv = 1.0 / v_vmem[...]

            @pl.loop(0, _BLOCK, step=_LANES, unroll=8)
            def _(c):
                o_vmem[pl.ds(c, _LANES)] = x_vmem[pl.ds(c, _LANES)] * inv

        pltpu.emit_pipeline(
            body,
            grid=(n // _BLOCK,),
            in_specs=[pl.BlockSpec((_BLOCK,), lambda i: (i,))],
            out_specs=[pl.BlockSpec((_BLOCK,), lambda i: (i,))],
            core_axis_name=("c", "s"),
            dimension_semantics=(pltpu.PARALLEL,),
        )(x_hbm, o_hbm)

    return _sc_scale(x1, v16).reshape(orig_shape)
